# baseline (speedup 1.0000x reference)
import functools
import math

import jax
import jax.numpy as jnp
from jax import lax
from jax.experimental import pallas as pl
from jax.experimental.pallas import tpu as pltpu
from jax.experimental.pallas import tpu_sc as plsc

_K = 8
_N = 32
_LANES = 16
_NC = 2
_NS = 16
_NW = _NC * _NS
_GROUPS_PER_W = 196
_G = _NW * _GROUPS_PER_W
_RPAD = _G * _LANES


def _sc_body(theta_hbm, u_hbm, marg_hbm, samp_hbm,
             th0, th1, uv0, uv1, wv0, wv1, bt0, bt1, mv0, mv1, sv0, sv1,
             sin0, sin1, sout0, sout1):
    wid = lax.axis_index("s") * _NC + lax.axis_index("c")
    lane = lax.iota(jnp.int32, _LANES)
    ones = jnp.full((_LANES,), 1.0, jnp.float32)
    zero = jnp.zeros((_LANES,), jnp.float32)
    base = wid * _GROUPS_PER_W

    th = (th0, th1)
    uv = (uv0, uv1)
    wv = (wv0, wv1)
    bt = (bt0, bt1)
    mv = (mv0, mv1)
    sv = (sv0, sv1)
    sin = (sin0, sin1)
    sout = (sout0, sout1)

    for btab in bt:
        for i in range(_N + 1):
            btab[i * (_K + 1)] = ones
            for j in range(min(_K, _N - i) + 1, _K + 1):
                btab[i * (_K + 1) + j] = zero

    def start_in(b, g):
        pltpu.async_copy(theta_hbm.at[g], th[b], sin[b])
        pltpu.async_copy(u_hbm.at[g], uv[b], sin[b])

    def wait_in(b, g):
        pltpu.make_async_copy(theta_hbm.at[g], th[b], sin[b]).wait()
        pltpu.make_async_copy(u_hbm.at[g], uv[b], sin[b]).wait()

    def start_out(b, g):
        pltpu.async_copy(mv[b], marg_hbm.at[g], sout[b])
        pltpu.async_copy(sv[b], samp_hbm.at[g], sout[b])

    def wait_out(b, g):
        pltpu.make_async_copy(mv[b], marg_hbm.at[g], sout[b]).wait()
        pltpu.make_async_copy(sv[b], samp_hbm.at[g], sout[b]).wait()

    start_in(0, base)
    start_in(1, base + 1)

    def compute_pair():
        B2 = (0, 1)
        for i in range(_N):
            for b in B2:
                wv[b][i] = jnp.exp(th[b][i])

        bs = [[ones] + [zero] * _K, [ones] + [zero] * _K]
        for i in range(_N - 1, -1, -1):
            wi = [wv[b][i] for b in B2]
            hi = min(_K, _N - i)
            for k in range(hi, 0, -1):
                for b in B2:
                    bs[b][k] = bs[b][k] + bs[b][k - 1] * wi[b]
            for j in range(1, hi + 1):
                for b in B2:
                    bt[b][i * (_K + 1) + j] = bs[b][j]

        fs = [[ones] + [zero] * _K, [ones] + [zero] * _K]
        for i in range(_N):
            wi = [wv[b][i] for b in B2]
            jlo = max(0, i - (_N - _K))
            jhi = min(i, _K - 1)
            num = [fs[b][jlo] * bt[b][(i + 1) * (_K + 1) + (_K - 1 - jlo)]
                   for b in B2]
            for j in range(jlo + 1, jhi + 1):
                for b in B2:
                    num[b] = num[b] + fs[b][j] * bt[b][(i + 1) * (_K + 1) + (_K - 1 - j)]
            for b in B2:
                mv[b][i] = wi[b] * num[b]
            hi = min(_K, i + 1)
            for k in range(hi, 0, -1):
                for b in B2:
                    fs[b][k] = fs[b][k] + fs[b][k - 1] * wi[b]
        inv = [1.0 / fs[b][_K] for b in B2]
        for i in range(_N):
            for b in B2:
                mv[b][i] = mv[b][i] * inv[b]

        r = [jnp.full((_LANES,), _K, jnp.int32) for _ in B2]
        for i in range(_N):
            g1 = [None, None]
            g2 = [None, None]
            for b in B2:
                g1[b] = plsc.load_gather(
                    bt[b], [(i + 1) * (_K + 1) + (r[b] - 1), lane])
                g2[b] = plsc.load_gather(
                    bt[b], [i * (_K + 1) + r[b], lane])
            for b in B2:
                wi = wv[b][i]
                ui = uv[b][i]
                take = (ui * g2[b] < wi * g1[b])
                if i > _N - _K:
                    take_edge = ui < jnp.minimum(wi, 1.0)
                    take = jnp.where(g2[b] == 0.0, take_edge, take)
                take = take & (r[b] > 0)
                sv[b][i] = take.astype(jnp.float32)
                r[b] = r[b] - take.astype(jnp.int32)

    def super_step(si, _):
        for b in range(2):
            g = base + 2 * si + b
            wait_in(b, g)

            @pl.when(si > 0)
            def _drain_out():
                wait_out(b, g)

        compute_pair()

        for b in range(2):
            g = base + 2 * si + b
            start_out(b, g)

            @pl.when(si < _GROUPS_PER_W // 2 - 1)
            def _prefetch():
                start_in(b, g + 2)
        return ()

    lax.fori_loop(0, _GROUPS_PER_W // 2, super_step, (), unroll=False)

    for b in range(2):
        wait_out(b, base + _GROUPS_PER_W - 2 + b)


@jax.jit
def kernel(scores):
    nnodes, choices, ensemble = scores.shape
    assert choices == _N and 2 ** int(math.log2(choices)) == choices
    rows = nnodes * ensemble
    theta = jnp.transpose(scores, (0, 2, 1)).reshape(rows, choices)

    u = jax.random.uniform(jax.random.key(42), (_N, 1, rows), dtype=theta.dtype)
    u2 = u[:, 0, :]

    pad = _RPAD - rows
    theta_p = jnp.pad(theta, ((0, pad), (0, 0)))
    u_p = jnp.pad(u2, ((0, 0), (0, pad)), constant_values=0.5)
    theta_b = theta_p.reshape(_G, _LANES, _N).transpose(0, 2, 1)
    u_b = u_p.reshape(_N, _G, _LANES).transpose(1, 0, 2)

    mesh = plsc.VectorSubcoreMesh(core_axis_name="c", subcore_axis_name="s",
                                  num_cores=_NC, num_subcores=_NS)
    marg_b, samp_b = pl.kernel(
        _sc_body,
        out_type=[
            jax.ShapeDtypeStruct((_G, _N, _LANES), jnp.float32),
            jax.ShapeDtypeStruct((_G, _N, _LANES), jnp.float32),
        ],
        mesh=mesh,
        compiler_params=pltpu.CompilerParams(needs_layout_passes=False),
        scratch_types=(
            [pltpu.VMEM((_N, _LANES), jnp.float32) for _ in range(2)]
            + [pltpu.VMEM((_N, _LANES), jnp.float32) for _ in range(2)]
            + [pltpu.VMEM((_N, _LANES), jnp.float32) for _ in range(2)]
            + [pltpu.VMEM(((_N + 1) * (_K + 1), _LANES), jnp.float32)
               for _ in range(2)]
            + [pltpu.VMEM((_N, _LANES), jnp.float32) for _ in range(2)]
            + [pltpu.VMEM((_N, _LANES), jnp.float32) for _ in range(2)]
            + [pltpu.SemaphoreType.DMA for _ in range(4)]
        ),
    )(theta_b, u_b)

    marg_flat = marg_b.transpose(0, 2, 1).reshape(_RPAD, _N)[:rows]
    samp_flat = samp_b.transpose(0, 2, 1).reshape(_RPAD, _N)[:rows]
    marginals = jnp.transpose(marg_flat.reshape(nnodes, ensemble, choices), (0, 2, 1))
    samples = jnp.transpose(samp_flat.reshape(nnodes, ensemble, choices), (0, 2, 1))[None]
    return samples, marginals


# speedup vs baseline: 1.0339x; 1.0339x over previous
import functools
import math

import jax
import jax.numpy as jnp
from jax import lax
from jax.experimental import pallas as pl
from jax.experimental.pallas import tpu as pltpu
from jax.experimental.pallas import tpu_sc as plsc

_K = 8
_N = 32
_LANES = 16
_NC = 2
_NS = 16
_NW = _NC * _NS
_GROUPS_PER_W = 196
_G = _NW * _GROUPS_PER_W
_RPAD = _G * _LANES


def _sc_body(theta_hbm, u_hbm, marg_hbm, samp_hbm,
             th0, th1, uv0, uv1, wv0, wv1, bt0, bt1, mv0, mv1, sv0, sv1,
             sin0, sin1, sout0, sout1):
    wid = lax.axis_index("s") * _NC + lax.axis_index("c")
    lane = lax.iota(jnp.int32, _LANES)
    ones = jnp.full((_LANES,), 1.0, jnp.float32)
    zero = jnp.zeros((_LANES,), jnp.float32)
    base = wid * _GROUPS_PER_W

    th = (th0, th1)
    uv = (uv0, uv1)
    wv = (wv0, wv1)
    bt = (bt0, bt1)
    mv = (mv0, mv1)
    sv = (sv0, sv1)
    sin = (sin0, sin1)
    sout = (sout0, sout1)

    for btab in bt:
        for i in range(_N + 1):
            btab[i * (_K + 1)] = ones
            for j in range(min(_K, _N - i) + 1, _K + 1):
                btab[i * (_K + 1) + j] = zero

    def start_in(b, g):
        pltpu.async_copy(theta_hbm.at[g], th[b], sin[b])
        pltpu.async_copy(u_hbm.at[g], uv[b], sin[b])

    def wait_in(b, g):
        pltpu.make_async_copy(theta_hbm.at[g], th[b], sin[b]).wait()
        pltpu.make_async_copy(u_hbm.at[g], uv[b], sin[b]).wait()

    def start_out(b, g):
        pltpu.async_copy(mv[b], marg_hbm.at[g], sout[b])
        pltpu.async_copy(sv[b], samp_hbm.at[g], sout[b])

    def wait_out(b, g):
        pltpu.make_async_copy(mv[b], marg_hbm.at[g], sout[b]).wait()
        pltpu.make_async_copy(sv[b], samp_hbm.at[g], sout[b]).wait()

    start_in(0, base)
    start_in(1, base + 1)

    def compute_pair():
        B2 = (0, 1)
        for b in B2:
            for i in range(_N):
                wv[b][i] = jnp.exp(th[b][i])

        for b in B2:
            bs = [ones] + [zero] * _K
            for i in range(_N - 1, -1, -1):
                wi = wv[b][i]
                hi = min(_K, _N - i)
                for k in range(hi, 0, -1):
                    bs[k] = bs[k] + bs[k - 1] * wi
                for j in range(1, hi + 1):
                    bt[b][i * (_K + 1) + j] = bs[j]

        for b in B2:
            fs = [ones] + [zero] * _K
            for i in range(_N):
                wi = wv[b][i]
                jlo = max(0, i - (_N - _K))
                jhi = min(i, _K - 1)
                num = fs[jlo] * bt[b][(i + 1) * (_K + 1) + (_K - 1 - jlo)]
                for j in range(jlo + 1, jhi + 1):
                    num = num + fs[j] * bt[b][(i + 1) * (_K + 1) + (_K - 1 - j)]
                mv[b][i] = wi * num
                hi = min(_K, i + 1)
                for k in range(hi, 0, -1):
                    fs[k] = fs[k] + fs[k - 1] * wi
            inv = 1.0 / fs[_K]
            for i in range(_N):
                mv[b][i] = mv[b][i] * inv

        r = [jnp.full((_LANES,), _K, jnp.int32) for _ in B2]
        for i in range(_N):
            g1 = [None, None]
            g2 = [None, None]
            for b in B2:
                g1[b] = plsc.load_gather(
                    bt[b], [(i + 1) * (_K + 1) + (r[b] - 1), lane])
                g2[b] = plsc.load_gather(
                    bt[b], [i * (_K + 1) + r[b], lane])
            for b in B2:
                wi = wv[b][i]
                ui = uv[b][i]
                take = (ui * g2[b] < wi * g1[b])
                if i > _N - _K:
                    take_edge = ui < jnp.minimum(wi, 1.0)
                    take = jnp.where(g2[b] == 0.0, take_edge, take)
                take = take & (r[b] > 0)
                sv[b][i] = take.astype(jnp.float32)
                r[b] = r[b] - take.astype(jnp.int32)

    def super_step(si, _):
        for b in range(2):
            g = base + 2 * si + b
            wait_in(b, g)

            @pl.when(si > 0)
            def _drain_out():
                wait_out(b, g)

        compute_pair()

        for b in range(2):
            g = base + 2 * si + b
            start_out(b, g)

            @pl.when(si < _GROUPS_PER_W // 2 - 1)
            def _prefetch():
                start_in(b, g + 2)
        return ()

    lax.fori_loop(0, _GROUPS_PER_W // 2, super_step, (), unroll=False)

    for b in range(2):
        wait_out(b, base + _GROUPS_PER_W - 2 + b)


@jax.jit
def kernel(scores):
    nnodes, choices, ensemble = scores.shape
    assert choices == _N and 2 ** int(math.log2(choices)) == choices
    rows = nnodes * ensemble
    theta = jnp.transpose(scores, (0, 2, 1)).reshape(rows, choices)

    u = jax.random.uniform(jax.random.key(42), (_N, 1, rows), dtype=theta.dtype)
    u2 = u[:, 0, :]

    pad = _RPAD - rows
    theta_p = jnp.pad(theta, ((0, pad), (0, 0)))
    u_p = jnp.pad(u2, ((0, 0), (0, pad)), constant_values=0.5)
    theta_b = theta_p.reshape(_G, _LANES, _N).transpose(0, 2, 1)
    u_b = u_p.reshape(_N, _G, _LANES).transpose(1, 0, 2)

    mesh = plsc.VectorSubcoreMesh(core_axis_name="c", subcore_axis_name="s",
                                  num_cores=_NC, num_subcores=_NS)
    marg_b, samp_b = pl.kernel(
        _sc_body,
        out_type=[
            jax.ShapeDtypeStruct((_G, _N, _LANES), jnp.float32),
            jax.ShapeDtypeStruct((_G, _N, _LANES), jnp.float32),
        ],
        mesh=mesh,
        compiler_params=pltpu.CompilerParams(needs_layout_passes=False),
        scratch_types=(
            [pltpu.VMEM((_N, _LANES), jnp.float32) for _ in range(2)]
            + [pltpu.VMEM((_N, _LANES), jnp.float32) for _ in range(2)]
            + [pltpu.VMEM((_N, _LANES), jnp.float32) for _ in range(2)]
            + [pltpu.VMEM(((_N + 1) * (_K + 1), _LANES), jnp.float32)
               for _ in range(2)]
            + [pltpu.VMEM((_N, _LANES), jnp.float32) for _ in range(2)]
            + [pltpu.VMEM((_N, _LANES), jnp.float32) for _ in range(2)]
            + [pltpu.SemaphoreType.DMA for _ in range(4)]
        ),
    )(theta_b, u_b)

    marg_flat = marg_b.transpose(0, 2, 1).reshape(_RPAD, _N)[:rows]
    samp_flat = samp_b.transpose(0, 2, 1).reshape(_RPAD, _N)[:rows]
    marginals = jnp.transpose(marg_flat.reshape(nnodes, ensemble, choices), (0, 2, 1))
    samples = jnp.transpose(samp_flat.reshape(nnodes, ensemble, choices), (0, 2, 1))[None]
    return samples, marginals


# speedup vs baseline: 1.1653x; 1.1272x over previous
import functools
import math

import jax
import jax.numpy as jnp
from jax import lax
from jax.experimental import pallas as pl
from jax.experimental.pallas import tpu as pltpu
from jax.experimental.pallas import tpu_sc as plsc

_K = 8
_N = 32
_LANES = 16
_NC = 2
_NS = 16
_NW = _NC * _NS
_GROUPS_PER_W = 196
_G = _NW * _GROUPS_PER_W
_RPAD = _G * _LANES


def _sc_body(theta_hbm, u_hbm, marg_hbm, samp_hbm,
             th0, th1, uv0, uv1, wv0, wv1, bt0, bt1, mv0, mv1, sv0, sv1,
             sin0, sin1, sout0, sout1):
    wid = lax.axis_index("s") * _NC + lax.axis_index("c")
    lane = lax.iota(jnp.int32, _LANES)
    ones = jnp.full((_LANES,), 1.0, jnp.float32)
    zero = jnp.zeros((_LANES,), jnp.float32)
    base = wid * _GROUPS_PER_W

    th = (th0, th1)
    uv = (uv0, uv1)
    wv = (wv0, wv1)
    bt = (bt0, bt1)
    mv = (mv0, mv1)
    sv = (sv0, sv1)
    sin = (sin0, sin1)
    sout = (sout0, sout1)

    for btab in bt:
        for i in range(_N + 1):
            btab[i * (_K + 1)] = ones
            for j in range(min(_K, _N - i) + 1, _K + 1):
                btab[i * (_K + 1) + j] = zero

    def start_in(b, g):
        pltpu.async_copy(theta_hbm.at[g], th[b], sin[b])
        pltpu.async_copy(u_hbm.at[g], uv[b], sin[b])

    def wait_in(b, g):
        pltpu.make_async_copy(theta_hbm.at[g], th[b], sin[b]).wait()
        pltpu.make_async_copy(u_hbm.at[g], uv[b], sin[b]).wait()

    def start_out(b, g):
        pltpu.async_copy(mv[b], marg_hbm.at[g], sout[b])
        pltpu.async_copy(sv[b], samp_hbm.at[g], sout[b])

    def wait_out(b, g):
        pltpu.make_async_copy(mv[b], marg_hbm.at[g], sout[b]).wait()
        pltpu.make_async_copy(sv[b], samp_hbm.at[g], sout[b]).wait()

    start_in(0, base)
    start_in(1, base + 1)

    def compute(b):
        theta_v, u_v, w_v, btab, marg_v, samp_v = \
            th[b], uv[b], wv[b], bt[b], mv[b], sv[b]

        for i in range(_N):
            w_v[i] = jnp.exp(theta_v[i])

        b = [ones] + [zero] * _K
        for i in range(_N - 1, -1, -1):
            wi = w_v[i]
            hi = min(_K, _N - i)
            for k in range(hi, 0, -1):
                b[k] = b[k] + b[k - 1] * wi
            for j in range(1, hi + 1):
                btab[i * (_K + 1) + j] = b[j]

        f = [ones] + [zero] * _K
        for i in range(_N):
            wi = w_v[i]
            jlo = max(0, i - (_N - _K))
            jhi = min(i, _K - 1)
            num = f[jlo] * btab[(i + 1) * (_K + 1) + (_K - 1 - jlo)]
            for j in range(jlo + 1, jhi + 1):
                num = num + f[j] * btab[(i + 1) * (_K + 1) + (_K - 1 - j)]
            marg_v[i] = wi * num
            hi = min(_K, i + 1)
            for k in range(hi, 0, -1):
                f[k] = f[k] + f[k - 1] * wi
        inv = 1.0 / f[_K]
        for i in range(_N):
            marg_v[i] = marg_v[i] * inv

        r = jnp.full((_LANES,), _K, jnp.int32)
        for i in range(_N):
            g1 = plsc.load_gather(btab, [(i + 1) * (_K + 1) + (r - 1), lane])
            g2 = plsc.load_gather(btab, [i * (_K + 1) + r, lane])
            wi = w_v[i]
            ui = u_v[i]
            take = ui * g2 < wi * g1
            if i > _N - _K:
                take = jnp.where(g2 == 0.0, ui < jnp.minimum(wi, 1.0), take)
            take = take & (r > 0)
            samp_v[i] = take.astype(jnp.float32)
            r = r - take.astype(jnp.int32)

    def super_step(si, _):
        for b in range(2):
            g = base + 2 * si + b
            wait_in(b, g)

            @pl.when(si > 0)
            def _drain_out():
                wait_out(b, g)

            compute(b)
            start_out(b, g)

            @pl.when(si < _GROUPS_PER_W // 2 - 1)
            def _prefetch():
                start_in(b, g + 2)
        return ()

    lax.fori_loop(0, _GROUPS_PER_W // 2, super_step, (), unroll=False)

    for b in range(2):
        wait_out(b, base + _GROUPS_PER_W - 2 + b)


@jax.jit
def kernel(scores):
    nnodes, choices, ensemble = scores.shape
    assert choices == _N and 2 ** int(math.log2(choices)) == choices
    rows = nnodes * ensemble
    theta = jnp.transpose(scores, (0, 2, 1)).reshape(rows, choices)

    u = jax.random.uniform(jax.random.key(42), (_N, 1, rows), dtype=theta.dtype)
    u2 = u[:, 0, :]

    pad = _RPAD - rows
    theta_p = jnp.pad(theta, ((0, pad), (0, 0)))
    u_p = jnp.pad(u2, ((0, 0), (0, pad)), constant_values=0.5)
    theta_b = theta_p.reshape(_G, _LANES, _N).transpose(0, 2, 1)
    u_b = u_p.reshape(_N, _G, _LANES).transpose(1, 0, 2)

    mesh = plsc.VectorSubcoreMesh(core_axis_name="c", subcore_axis_name="s",
                                  num_cores=_NC, num_subcores=_NS)
    marg_b, samp_b = pl.kernel(
        _sc_body,
        out_type=[
            jax.ShapeDtypeStruct((_G, _N, _LANES), jnp.float32),
            jax.ShapeDtypeStruct((_G, _N, _LANES), jnp.float32),
        ],
        mesh=mesh,
        compiler_params=pltpu.CompilerParams(needs_layout_passes=False),
        scratch_types=(
            [pltpu.VMEM((_N, _LANES), jnp.float32) for _ in range(2)]
            + [pltpu.VMEM((_N, _LANES), jnp.float32) for _ in range(2)]
            + [pltpu.VMEM((_N, _LANES), jnp.float32) for _ in range(2)]
            + [pltpu.VMEM(((_N + 1) * (_K + 1), _LANES), jnp.float32)
               for _ in range(2)]
            + [pltpu.VMEM((_N, _LANES), jnp.float32) for _ in range(2)]
            + [pltpu.VMEM((_N, _LANES), jnp.float32) for _ in range(2)]
            + [pltpu.SemaphoreType.DMA for _ in range(4)]
        ),
    )(theta_b, u_b)

    marg_flat = marg_b.transpose(0, 2, 1).reshape(_RPAD, _N)[:rows]
    samp_flat = samp_b.transpose(0, 2, 1).reshape(_RPAD, _N)[:rows]
    marginals = jnp.transpose(marg_flat.reshape(nnodes, ensemble, choices), (0, 2, 1))
    samples = jnp.transpose(samp_flat.reshape(nnodes, ensemble, choices), (0, 2, 1))[None]
    return samples, marginals
